# Initial kernel scaffold; baseline (speedup 1.0000x reference)
import jax
import jax.numpy as jnp
from jax.experimental import pallas as pl


def kernel(hidden_states, router_logits, w13_weight, w2_weight):
    raise NotImplementedError("write your pallas kernel here")


# speedup vs baseline: 1.7029x; 1.7029x over previous
import functools

import jax
import jax.numpy as jnp
from jax import lax
from jax.experimental import pallas as pl
from jax.experimental.pallas import tpu as pltpu
from jax.experimental.pallas import tpu_sc as plsc

E = 8
K = 2
T = 4096
H = 1024
I = 4096
BM = 256
NB = T * K // BM + E
S_PAD = NB * BM
C = 512
NJ = I // C
NBPAD = 64
CH = 512
D_SUB = 64
C_SUB = 32


def _routing_body(logits_ref, slot_ref, wrep_ref, bexp_ref):
    logits = logits_ref[...]
    m = jnp.max(logits, axis=1, keepdims=True)
    ex = jnp.exp(logits - m)
    p = ex / jnp.sum(ex, axis=1, keepdims=True)
    ei = lax.broadcasted_iota(jnp.int32, (T, E), 1)
    p1 = jnp.max(p, axis=1, keepdims=True)
    id1 = jnp.min(jnp.where(p == p1, ei, E), axis=1, keepdims=True)
    oh1 = (ei == id1).astype(jnp.float32)
    pm = jnp.where(ei == id1, -1.0, p)
    p2 = jnp.max(pm, axis=1, keepdims=True)
    id2 = jnp.min(jnp.where(pm == p2, ei, E), axis=1, keepdims=True)
    oh2 = (ei == id2).astype(jnp.float32)
    wsum = p1 + p2
    w1 = p1 / wsum
    w2 = p2 / wsum

    oh_tok = oh1 + oh2
    stril = (lax.broadcasted_iota(jnp.int32, (CH, CH), 1)
             < lax.broadcasted_iota(jnp.int32, (CH, CH), 0)).astype(jnp.float32)
    base = jnp.zeros((1, E), jnp.float32)
    prefs = []
    for ci in range(T // CH):
        blk = oh_tok[ci * CH:(ci + 1) * CH, :]
        within = lax.dot_general(stril, blk, (((1,), (0,)), ((), ())),
                                 preferred_element_type=jnp.float32)
        prefs.append(within + base)
        base = base + jnp.sum(blk, axis=0, keepdims=True)
    prefix = jnp.concatenate(prefs, axis=0)

    counts = base
    blocks = jnp.floor((counts + (BM - 1)) / BM)
    tri = (lax.broadcasted_iota(jnp.int32, (E, E), 0)
           <= lax.broadcasted_iota(jnp.int32, (E, E), 1)).astype(jnp.float32)
    cum = lax.dot_general(blocks, tri, (((1,), (0,)), ((), ())),
                          preferred_element_type=jnp.float32)
    offset = (cum - blocks) * BM

    r1 = jnp.sum(prefix * oh1, axis=1)
    r2 = jnp.sum(prefix * oh2, axis=1)
    o1 = jnp.sum(offset * oh1, axis=1)
    o2 = jnp.sum(offset * oh2, axis=1)
    slot1 = (o1 + r1).astype(jnp.int32)
    slot2 = (o2 + r2).astype(jnp.int32)
    slot_ref[...] = jnp.concatenate(
        [slot1.reshape(1, T), slot2.reshape(1, T)], axis=0)
    wrep_ref[...] = jnp.concatenate(
        [jnp.broadcast_to(w1, (T, 128)), jnp.broadcast_to(w2, (T, 128))],
        axis=0)

    bi = lax.broadcasted_iota(jnp.int32, (NBPAD, E), 0).astype(jnp.float32)
    be = jnp.sum((bi >= jnp.broadcast_to(cum, (NBPAD, E))).astype(jnp.int32),
                 axis=1)
    bexp_ref[...] = jnp.broadcast_to(
        jnp.minimum(be, E - 1).reshape(1, NBPAD), (8, NBPAD))


def _routing(router_logits):
    return pl.pallas_call(
        _routing_body,
        out_shape=(
            jax.ShapeDtypeStruct((K, T), jnp.int32),
            jax.ShapeDtypeStruct((K * T, 128), jnp.float32),
            jax.ShapeDtypeStruct((8, NBPAD), jnp.int32),
        ),
    )(router_logits)


def _dispatch_body(hidden_hbm, slot_hbm, wrep_hbm, xpad_hbm, wb_hbm,
                   idx_v, rows_v, wrows_v, sem):
    c = lax.axis_index("c")
    s = lax.axis_index("s")
    wid = s * 2 + c
    k = wid % 2
    part = wid // 2
    per_part = T // 16
    for u in range(per_part // D_SUB):
        tbase = part * per_part + u * D_SUB
        pltpu.sync_copy(slot_hbm.at[pl.ds(k * T + tbase, D_SUB)], idx_v)
        pltpu.sync_copy(hidden_hbm.at[pl.ds(tbase, D_SUB)], rows_v)
        pltpu.async_copy(rows_v, xpad_hbm.at[idx_v], sem).wait()
        pltpu.sync_copy(wrep_hbm.at[pl.ds(k * T + tbase, D_SUB)], wrows_v)
        pltpu.async_copy(wrows_v, wb_hbm.at[idx_v], sem).wait()


def _gmm_body(bexp_ref, x_ref, wg_ref, wu_ref, w2_ref, wb_ref, y_ref):
    j = pl.program_id(0)
    b = pl.program_id(1)
    x = x_ref[...]
    gate = lax.dot_general(x, wg_ref[0], (((1,), (1,)), ((), ())),
                           preferred_element_type=jnp.float32)
    up = lax.dot_general(x, wu_ref[0], (((1,), (1,)), ((), ())),
                         preferred_element_type=jnp.float32)
    act = gate * jax.nn.sigmoid(gate) * up
    y = lax.dot_general(act, w2_ref[0], (((1,), (1,)), ((), ())),
                        preferred_element_type=jnp.float32)
    y = y * wb_ref[:, 0:1]
    rows = pl.ds(b * BM, BM)

    @pl.when(j == 0)
    def _():
        y_ref[rows, :] = y

    @pl.when(j != 0)
    def _():
        y_ref[rows, :] = y_ref[rows, :] + y


def _gmm(bexp, x_pad, w13, w2, w_bcast):
    grid_spec = pltpu.PrefetchScalarGridSpec(
        num_scalar_prefetch=1,
        grid=(NJ, NB),
        in_specs=[
            pl.BlockSpec((BM, H), lambda j, b, be: (b, 0)),
            pl.BlockSpec((1, C, H), lambda j, b, be: (be[0, b], j, 0)),
            pl.BlockSpec((1, C, H), lambda j, b, be: (be[0, b], NJ + j, 0)),
            pl.BlockSpec((1, H, C), lambda j, b, be: (be[0, b], 0, j)),
            pl.BlockSpec((BM, 128), lambda j, b, be: (b, 0)),
        ],
        out_specs=pl.BlockSpec((S_PAD, H), lambda j, b, be: (0, 0)),
    )
    return pl.pallas_call(
        _gmm_body,
        grid_spec=grid_spec,
        out_shape=jax.ShapeDtypeStruct((S_PAD, H), jnp.float32),
        compiler_params=pltpu.CompilerParams(
            dimension_semantics=("arbitrary", "arbitrary")),
    )(bexp, x_pad, w13, w13, w2, w_bcast)


def _combine_body(y_hbm, slot_hbm, out_hbm, i0_v, i1_v, a_v, b_v, sem0, sem1):
    c = lax.axis_index("c")
    s = lax.axis_index("s")
    wid = s * 2 + c
    per_tile = T // 32
    for u in range(per_tile // C_SUB):
        tbase = wid * per_tile + u * C_SUB
        pltpu.sync_copy(slot_hbm.at[pl.ds(tbase, C_SUB)], i0_v)
        pltpu.sync_copy(slot_hbm.at[pl.ds(T + tbase, C_SUB)], i1_v)
        cp0 = pltpu.async_copy(y_hbm.at[i0_v], a_v, sem0)
        cp1 = pltpu.async_copy(y_hbm.at[i1_v], b_v, sem1)
        cp0.wait()
        cp1.wait()

        def row_body(r, _):
            def col_body(ci, _2):
                off = pl.multiple_of(ci * 16, 16)
                a_v[r, pl.ds(off, 16)] = (a_v[r, pl.ds(off, 16)]
                                          + b_v[r, pl.ds(off, 16)])
                return 0
            return lax.fori_loop(0, H // 16, col_body, 0)

        lax.fori_loop(0, C_SUB, row_body, 0)
        pltpu.sync_copy(a_v, out_hbm.at[pl.ds(tbase, C_SUB)])


@functools.cache
def _sc_kernels():
    mesh = plsc.VectorSubcoreMesh(core_axis_name="c", subcore_axis_name="s")
    dispatch = pl.kernel(
        _dispatch_body,
        out_type=(jax.ShapeDtypeStruct((S_PAD, H), jnp.float32),
                  jax.ShapeDtypeStruct((S_PAD, 128), jnp.float32)),
        mesh=mesh,
        scratch_types=[pltpu.VMEM((D_SUB,), jnp.int32),
                       pltpu.VMEM((D_SUB, H), jnp.float32),
                       pltpu.VMEM((D_SUB, 128), jnp.float32),
                       pltpu.SemaphoreType.DMA],
    )
    combine = pl.kernel(
        _combine_body,
        out_type=jax.ShapeDtypeStruct((T, H), jnp.float32),
        mesh=mesh,
        scratch_types=[pltpu.VMEM((C_SUB,), jnp.int32),
                       pltpu.VMEM((C_SUB,), jnp.int32),
                       pltpu.VMEM((C_SUB, H), jnp.float32),
                       pltpu.VMEM((C_SUB, H), jnp.float32),
                       pltpu.SemaphoreType.DMA,
                       pltpu.SemaphoreType.DMA],
    )
    return dispatch, combine


def kernel(hidden_states, router_logits, w13_weight, w2_weight):
    dispatch, combine = _sc_kernels()
    slot, wrep, bexp8 = _routing(router_logits)
    slot_flat = slot.reshape(K * T)
    x_pad, w_bcast = dispatch(hidden_states, slot_flat, wrep)
    y_pad = _gmm(bexp8, x_pad, w13_weight, w2_weight, w_bcast)
    return combine(y_pad, slot_flat)


# speedup vs baseline: 2.0085x; 1.1795x over previous
import functools

import jax
import jax.numpy as jnp
from jax import lax
from jax.experimental import pallas as pl
from jax.experimental.pallas import tpu as pltpu
from jax.experimental.pallas import tpu_sc as plsc

E = 8
K = 2
T = 4096
H = 1024
I = 4096
BM = 256
NB = T * K // BM + E
NG = 4
NB2 = NB // NG
S_PAD = NB * BM
C = 1024
NJ = I // C
NBPAD = 64
CH = 512
D_SUB = 64
C_SUB = 32


def _routing_body(logits_ref, slot_ref, wrep_ref, bexp_ref):
    logits = logits_ref[...]
    m = jnp.max(logits, axis=1, keepdims=True)
    ex = jnp.exp(logits - m)
    p = ex / jnp.sum(ex, axis=1, keepdims=True)
    ei = lax.broadcasted_iota(jnp.int32, (T, E), 1)
    p1 = jnp.max(p, axis=1, keepdims=True)
    id1 = jnp.min(jnp.where(p == p1, ei, E), axis=1, keepdims=True)
    oh1 = (ei == id1).astype(jnp.float32)
    pm = jnp.where(ei == id1, -1.0, p)
    p2 = jnp.max(pm, axis=1, keepdims=True)
    id2 = jnp.min(jnp.where(pm == p2, ei, E), axis=1, keepdims=True)
    oh2 = (ei == id2).astype(jnp.float32)
    wsum = p1 + p2
    w1 = p1 / wsum
    w2 = p2 / wsum

    oh_tok = oh1 + oh2
    stril = (lax.broadcasted_iota(jnp.int32, (CH, CH), 1)
             < lax.broadcasted_iota(jnp.int32, (CH, CH), 0)).astype(jnp.float32)
    base = jnp.zeros((1, E), jnp.float32)
    prefs = []
    for ci in range(T // CH):
        blk = oh_tok[ci * CH:(ci + 1) * CH, :]
        within = lax.dot_general(stril, blk, (((1,), (0,)), ((), ())),
                                 preferred_element_type=jnp.float32)
        prefs.append(within + base)
        base = base + jnp.sum(blk, axis=0, keepdims=True)
    prefix = jnp.concatenate(prefs, axis=0)

    counts = base
    blocks = jnp.floor((counts + (BM - 1)) / BM)
    tri = (lax.broadcasted_iota(jnp.int32, (E, E), 0)
           <= lax.broadcasted_iota(jnp.int32, (E, E), 1)).astype(jnp.float32)
    cum = lax.dot_general(blocks, tri, (((1,), (0,)), ((), ())),
                          preferred_element_type=jnp.float32)
    offset = (cum - blocks) * BM

    r1 = jnp.sum(prefix * oh1, axis=1)
    r2 = jnp.sum(prefix * oh2, axis=1)
    o1 = jnp.sum(offset * oh1, axis=1)
    o2 = jnp.sum(offset * oh2, axis=1)
    slot1 = (o1 + r1).astype(jnp.int32)
    slot2 = (o2 + r2).astype(jnp.int32)
    slot_ref[...] = jnp.concatenate(
        [slot1.reshape(1, T), slot2.reshape(1, T)], axis=0)
    wrep_ref[...] = jnp.concatenate(
        [jnp.broadcast_to(w1, (T, 128)), jnp.broadcast_to(w2, (T, 128))],
        axis=0)

    bi = lax.broadcasted_iota(jnp.int32, (NBPAD, E), 0).astype(jnp.float32)
    be = jnp.sum((bi >= jnp.broadcast_to(cum, (NBPAD, E))).astype(jnp.int32),
                 axis=1)
    bexp_ref[...] = jnp.broadcast_to(
        jnp.minimum(be, E - 1).reshape(1, NBPAD), (8, NBPAD))


def _routing(router_logits):
    return pl.pallas_call(
        _routing_body,
        out_shape=(
            jax.ShapeDtypeStruct((K, T), jnp.int32),
            jax.ShapeDtypeStruct((K * T, 128), jnp.float32),
            jax.ShapeDtypeStruct((8, NBPAD), jnp.int32),
        ),
    )(router_logits)


def _dispatch_body(hidden_hbm, slot_hbm, wrep_hbm, xpad_hbm, wb_hbm,
                   idx_v, rows_v, wrows_v, sem, sem2):
    c = lax.axis_index("c")
    s = lax.axis_index("s")
    wid = s * 2 + c
    k = wid % 2
    part = wid // 2
    per_part = T // 16
    for u in range(per_part // D_SUB):
        tbase = part * per_part + u * D_SUB
        pltpu.sync_copy(slot_hbm.at[pl.ds(k * T + tbase, D_SUB)], idx_v)
        pltpu.sync_copy(hidden_hbm.at[pl.ds(tbase, D_SUB)], rows_v)
        cp1 = pltpu.async_copy(rows_v, xpad_hbm.at[idx_v], sem)
        pltpu.sync_copy(wrep_hbm.at[pl.ds(k * T + tbase, D_SUB)], wrows_v)
        cp2 = pltpu.async_copy(wrows_v, wb_hbm.at[idx_v], sem2)
        cp1.wait()
        cp2.wait()


def _gmm_body(bexp_ref, x_ref, wg_ref, wu_ref, w2_ref, wb_ref, y_ref):
    j = pl.program_id(1)
    b = pl.program_id(2)
    x = x_ref[...]
    gate = lax.dot_general(x, wg_ref[0], (((1,), (1,)), ((), ())),
                           preferred_element_type=jnp.float32)
    up = lax.dot_general(x, wu_ref[0], (((1,), (1,)), ((), ())),
                         preferred_element_type=jnp.float32)
    act = gate * jax.nn.sigmoid(gate) * up
    y = lax.dot_general(act, w2_ref[0], (((1,), (1,)), ((), ())),
                        preferred_element_type=jnp.float32)
    y = y * wb_ref[:, 0:1]
    rows = pl.ds(b * BM, BM)

    @pl.when(j == 0)
    def _():
        y_ref[rows, :] = y

    @pl.when(j != 0)
    def _():
        y_ref[rows, :] = y_ref[rows, :] + y


def _gmm(bexp, x_pad, w13, w2, w_bcast):
    grid_spec = pltpu.PrefetchScalarGridSpec(
        num_scalar_prefetch=1,
        grid=(NG, NJ, NB2),
        in_specs=[
            pl.BlockSpec((BM, H), lambda g, j, b, be: (g * NB2 + b, 0)),
            pl.BlockSpec((1, C, H),
                         lambda g, j, b, be: (be[0, g * NB2 + b], j, 0)),
            pl.BlockSpec((1, C, H),
                         lambda g, j, b, be: (be[0, g * NB2 + b], NJ + j, 0)),
            pl.BlockSpec((1, H, C),
                         lambda g, j, b, be: (be[0, g * NB2 + b], 0, j)),
            pl.BlockSpec((BM, 128), lambda g, j, b, be: (g * NB2 + b, 0)),
        ],
        out_specs=pl.BlockSpec((S_PAD // NG, H), lambda g, j, b, be: (g, 0)),
    )
    return pl.pallas_call(
        _gmm_body,
        grid_spec=grid_spec,
        out_shape=jax.ShapeDtypeStruct((S_PAD, H), jnp.float32),
        compiler_params=pltpu.CompilerParams(
            dimension_semantics=("arbitrary", "arbitrary", "arbitrary")),
    )(bexp, x_pad, w13, w13, w2, w_bcast)


def _combine_body(y_hbm, slot_hbm, out_hbm, i0_v, i1_v, a_v, b_v, sem0, sem1):
    c = lax.axis_index("c")
    s = lax.axis_index("s")
    wid = s * 2 + c
    per_tile = T // 32
    for u in range(per_tile // C_SUB):
        tbase = wid * per_tile + u * C_SUB
        pltpu.sync_copy(slot_hbm.at[pl.ds(tbase, C_SUB)], i0_v)
        pltpu.sync_copy(slot_hbm.at[pl.ds(T + tbase, C_SUB)], i1_v)
        cp0 = pltpu.async_copy(y_hbm.at[i0_v], a_v, sem0)
        cp1 = pltpu.async_copy(y_hbm.at[i1_v], b_v, sem1)
        cp0.wait()
        cp1.wait()

        def row_body(r, _):
            def col_body(ci, _2):
                for q in range(8):
                    off = pl.multiple_of(ci * 128 + q * 16, 16)
                    plsc.addupdate(a_v.at[r, pl.ds(off, 16)],
                                   b_v[r, pl.ds(off, 16)])
                return 0
            return lax.fori_loop(0, H // 128, col_body, 0)

        lax.fori_loop(0, C_SUB, row_body, 0)
        pltpu.sync_copy(a_v, out_hbm.at[pl.ds(tbase, C_SUB)])


@functools.cache
def _sc_kernels():
    mesh = plsc.VectorSubcoreMesh(core_axis_name="c", subcore_axis_name="s")
    dispatch = pl.kernel(
        _dispatch_body,
        out_type=(jax.ShapeDtypeStruct((S_PAD, H), jnp.float32),
                  jax.ShapeDtypeStruct((S_PAD, 128), jnp.float32)),
        mesh=mesh,
        scratch_types=[pltpu.VMEM((D_SUB,), jnp.int32),
                       pltpu.VMEM((D_SUB, H), jnp.float32),
                       pltpu.VMEM((D_SUB, 128), jnp.float32),
                       pltpu.SemaphoreType.DMA,
                       pltpu.SemaphoreType.DMA],
    )
    combine = pl.kernel(
        _combine_body,
        out_type=jax.ShapeDtypeStruct((T, H), jnp.float32),
        mesh=mesh,
        scratch_types=[pltpu.VMEM((C_SUB,), jnp.int32),
                       pltpu.VMEM((C_SUB,), jnp.int32),
                       pltpu.VMEM((C_SUB, H), jnp.float32),
                       pltpu.VMEM((C_SUB, H), jnp.float32),
                       pltpu.SemaphoreType.DMA,
                       pltpu.SemaphoreType.DMA],
    )
    return dispatch, combine


def kernel(hidden_states, router_logits, w13_weight, w2_weight):
    dispatch, combine = _sc_kernels()
    slot, wrep, bexp8 = _routing(router_logits)
    slot_flat = slot.reshape(K * T)
    x_pad, w_bcast = dispatch(hidden_states, slot_flat, wrep)
    y_pad = _gmm(bexp8, x_pad, w13_weight, w2_weight, w_bcast)
    return combine(y_pad, slot_flat)


# speedup vs baseline: 2.0132x; 1.0023x over previous
import functools

import jax
import jax.numpy as jnp
from jax import lax
from jax.experimental import pallas as pl
from jax.experimental.pallas import tpu as pltpu
from jax.experimental.pallas import tpu_sc as plsc

E = 8
K = 2
T = 4096
H = 1024
I = 4096
BM = 256
NB = T * K // BM + E
NG = 4
NB2 = NB // NG
S_PAD = NB * BM
C = 1024
NJ = I // C
NBPAD = 64
CH = 512
D_SUB = 32
C_SUB = 16


def _routing_body(logits_ref, slot_ref, wrep_ref, bexp_ref):
    logits = logits_ref[...]
    m = jnp.max(logits, axis=1, keepdims=True)
    ex = jnp.exp(logits - m)
    p = ex / jnp.sum(ex, axis=1, keepdims=True)
    ei = lax.broadcasted_iota(jnp.int32, (T, E), 1)
    p1 = jnp.max(p, axis=1, keepdims=True)
    id1 = jnp.min(jnp.where(p == p1, ei, E), axis=1, keepdims=True)
    oh1 = (ei == id1).astype(jnp.float32)
    pm = jnp.where(ei == id1, -1.0, p)
    p2 = jnp.max(pm, axis=1, keepdims=True)
    id2 = jnp.min(jnp.where(pm == p2, ei, E), axis=1, keepdims=True)
    oh2 = (ei == id2).astype(jnp.float32)
    wsum = p1 + p2
    w1 = p1 / wsum
    w2 = p2 / wsum

    oh_tok = oh1 + oh2
    stril = (lax.broadcasted_iota(jnp.int32, (CH, CH), 1)
             < lax.broadcasted_iota(jnp.int32, (CH, CH), 0)).astype(jnp.float32)
    base = jnp.zeros((1, E), jnp.float32)
    prefs = []
    for ci in range(T // CH):
        blk = oh_tok[ci * CH:(ci + 1) * CH, :]
        within = lax.dot_general(stril, blk, (((1,), (0,)), ((), ())),
                                 preferred_element_type=jnp.float32)
        prefs.append(within + base)
        base = base + jnp.sum(blk, axis=0, keepdims=True)
    prefix = jnp.concatenate(prefs, axis=0)

    counts = base
    blocks = jnp.floor((counts + (BM - 1)) / BM)
    tri = (lax.broadcasted_iota(jnp.int32, (E, E), 0)
           <= lax.broadcasted_iota(jnp.int32, (E, E), 1)).astype(jnp.float32)
    cum = lax.dot_general(blocks, tri, (((1,), (0,)), ((), ())),
                          preferred_element_type=jnp.float32)
    offset = (cum - blocks) * BM

    r1 = jnp.sum(prefix * oh1, axis=1)
    r2 = jnp.sum(prefix * oh2, axis=1)
    o1 = jnp.sum(offset * oh1, axis=1)
    o2 = jnp.sum(offset * oh2, axis=1)
    slot1 = (o1 + r1).astype(jnp.int32)
    slot2 = (o2 + r2).astype(jnp.int32)
    slot_ref[...] = jnp.concatenate(
        [slot1.reshape(1, T), slot2.reshape(1, T)], axis=0)
    wrep_ref[...] = jnp.concatenate(
        [jnp.broadcast_to(w1, (T, 128)), jnp.broadcast_to(w2, (T, 128))],
        axis=0)

    bi = lax.broadcasted_iota(jnp.int32, (NBPAD, E), 0).astype(jnp.float32)
    be = jnp.sum((bi >= jnp.broadcast_to(cum, (NBPAD, E))).astype(jnp.int32),
                 axis=1)
    bexp_ref[...] = jnp.broadcast_to(
        jnp.minimum(be, E - 1).reshape(1, NBPAD), (8, NBPAD))


def _routing(router_logits):
    return pl.pallas_call(
        _routing_body,
        out_shape=(
            jax.ShapeDtypeStruct((K, T), jnp.int32),
            jax.ShapeDtypeStruct((K * T, 128), jnp.float32),
            jax.ShapeDtypeStruct((8, NBPAD), jnp.int32),
        ),
    )(router_logits)


def _dispatch_body(hidden_hbm, slot_hbm, wrep_hbm, xpad_hbm, wb_hbm,
                   idx_v, rows_v, wrows_v, s0, s1, s2, s3):
    sem = [s0, s1]
    sem2 = [s2, s3]
    c = lax.axis_index("c")
    s = lax.axis_index("s")
    wid = s * 2 + c
    k = wid % 2
    part = wid // 2
    per_part = T // 16
    n_u = per_part // D_SUB
    pend = [None, None]
    for u in range(n_u):
        b = u % 2
        if pend[b] is not None:
            pend[b][0].wait()
            pend[b][1].wait()
        tbase = part * per_part + u * D_SUB
        pltpu.sync_copy(slot_hbm.at[pl.ds(k * T + tbase, D_SUB)], idx_v.at[b])
        pltpu.sync_copy(hidden_hbm.at[pl.ds(tbase, D_SUB)], rows_v.at[b])
        pltpu.sync_copy(wrep_hbm.at[pl.ds(k * T + tbase, D_SUB)], wrows_v.at[b])
        cp1 = pltpu.async_copy(rows_v.at[b], xpad_hbm.at[idx_v.at[b]], sem[b])
        cp2 = pltpu.async_copy(wrows_v.at[b], wb_hbm.at[idx_v.at[b]], sem2[b])
        pend[b] = (cp1, cp2)
    for b in range(2):
        if pend[b] is not None:
            pend[b][0].wait()
            pend[b][1].wait()


def _gmm_body(bexp_ref, x_ref, wg_ref, wu_ref, w2_ref, wb_ref, y_ref):
    j = pl.program_id(1)
    b = pl.program_id(2)
    x = x_ref[...]
    gate = lax.dot_general(x, wg_ref[0], (((1,), (1,)), ((), ())),
                           preferred_element_type=jnp.float32)
    up = lax.dot_general(x, wu_ref[0], (((1,), (1,)), ((), ())),
                         preferred_element_type=jnp.float32)
    act = gate * jax.nn.sigmoid(gate) * up
    y = lax.dot_general(act, w2_ref[0], (((1,), (1,)), ((), ())),
                        preferred_element_type=jnp.float32)
    y = y * wb_ref[:, 0:1]
    rows = pl.ds(b * BM, BM)

    @pl.when(j == 0)
    def _():
        y_ref[rows, :] = y

    @pl.when(j != 0)
    def _():
        y_ref[rows, :] = y_ref[rows, :] + y


def _gmm(bexp, x_pad, w13, w2, w_bcast):
    grid_spec = pltpu.PrefetchScalarGridSpec(
        num_scalar_prefetch=1,
        grid=(NG, NJ, NB2),
        in_specs=[
            pl.BlockSpec((BM, H), lambda g, j, b, be: (g * NB2 + b, 0)),
            pl.BlockSpec((1, C, H),
                         lambda g, j, b, be: (be[0, g * NB2 + b], j, 0)),
            pl.BlockSpec((1, C, H),
                         lambda g, j, b, be: (be[0, g * NB2 + b], NJ + j, 0)),
            pl.BlockSpec((1, H, C),
                         lambda g, j, b, be: (be[0, g * NB2 + b], 0, j)),
            pl.BlockSpec((BM, 128), lambda g, j, b, be: (g * NB2 + b, 0)),
        ],
        out_specs=pl.BlockSpec((S_PAD // NG, H), lambda g, j, b, be: (g, 0)),
    )
    return pl.pallas_call(
        _gmm_body,
        grid_spec=grid_spec,
        out_shape=jax.ShapeDtypeStruct((S_PAD, H), jnp.float32),
        compiler_params=pltpu.CompilerParams(
            dimension_semantics=("arbitrary", "arbitrary", "arbitrary")),
    )(bexp, x_pad, w13, w13, w2, w_bcast)


def _combine_body(y_hbm, slot_hbm, out_hbm, i0_v, i1_v, a_v, b_v,
                  sg0, sg1, sg2, sg3, sw0, sw1):
    semg = [(sg0, sg1), (sg2, sg3)]
    semw = [sw0, sw1]
    c = lax.axis_index("c")
    s = lax.axis_index("s")
    wid = s * 2 + c
    per_tile = T // 32
    n_u = per_tile // C_SUB

    def issue(u, b):
        tbase = wid * per_tile + u * C_SUB
        pltpu.sync_copy(slot_hbm.at[pl.ds(tbase, C_SUB)], i0_v.at[b])
        pltpu.sync_copy(slot_hbm.at[pl.ds(T + tbase, C_SUB)], i1_v.at[b])
        cp0 = pltpu.async_copy(y_hbm.at[i0_v.at[b]], a_v.at[b], semg[b][0])
        cp1 = pltpu.async_copy(y_hbm.at[i1_v.at[b]], b_v.at[b], semg[b][1])
        return cp0, cp1

    pend_g = [None, None]
    pend_w = [None, None]
    pend_g[0] = issue(0, 0)
    for u in range(n_u):
        b = u % 2
        pend_g[b][0].wait()
        pend_g[b][1].wait()
        nb = 1 - b
        if u + 1 < n_u:
            if pend_w[nb] is not None:
                pend_w[nb].wait()
                pend_w[nb] = None
            pend_g[nb] = issue(u + 1, nb)

        def row_body(r, _):
            def col_body(ci, _2):
                for q in range(8):
                    off = pl.multiple_of(ci * 128 + q * 16, 16)
                    plsc.addupdate(a_v.at[b, r, pl.ds(off, 16)],
                                   b_v[b, r, pl.ds(off, 16)])
                return 0
            return lax.fori_loop(0, H // 128, col_body, 0)

        lax.fori_loop(0, C_SUB, row_body, 0)
        tbase = wid * per_tile + u * C_SUB
        pend_w[b] = pltpu.async_copy(a_v.at[b], out_hbm.at[pl.ds(tbase, C_SUB)],
                                     semw[b])
    for b in range(2):
        if pend_w[b] is not None:
            pend_w[b].wait()


@functools.cache
def _sc_kernels():
    mesh = plsc.VectorSubcoreMesh(core_axis_name="c", subcore_axis_name="s")
    dispatch = pl.kernel(
        _dispatch_body,
        out_type=(jax.ShapeDtypeStruct((S_PAD, H), jnp.float32),
                  jax.ShapeDtypeStruct((S_PAD, 128), jnp.float32)),
        mesh=mesh,
        scratch_types=[pltpu.VMEM((2, D_SUB), jnp.int32),
                       pltpu.VMEM((2, D_SUB, H), jnp.float32),
                       pltpu.VMEM((2, D_SUB, 128), jnp.float32),
                       pltpu.SemaphoreType.DMA,
                       pltpu.SemaphoreType.DMA,
                       pltpu.SemaphoreType.DMA,
                       pltpu.SemaphoreType.DMA],
    )
    combine = pl.kernel(
        _combine_body,
        out_type=jax.ShapeDtypeStruct((T, H), jnp.float32),
        mesh=mesh,
        scratch_types=[pltpu.VMEM((2, C_SUB), jnp.int32),
                       pltpu.VMEM((2, C_SUB), jnp.int32),
                       pltpu.VMEM((2, C_SUB, H), jnp.float32),
                       pltpu.VMEM((2, C_SUB, H), jnp.float32),
                       pltpu.SemaphoreType.DMA,
                       pltpu.SemaphoreType.DMA,
                       pltpu.SemaphoreType.DMA,
                       pltpu.SemaphoreType.DMA,
                       pltpu.SemaphoreType.DMA,
                       pltpu.SemaphoreType.DMA],
    )
    return dispatch, combine


def kernel(hidden_states, router_logits, w13_weight, w2_weight):
    dispatch, combine = _sc_kernels()
    slot, wrep, bexp8 = _routing(router_logits)
    slot_flat = slot.reshape(K * T)
    x_pad, w_bcast = dispatch(hidden_states, slot_flat, wrep)
    y_pad = _gmm(bexp8, x_pad, w13_weight, w2_weight, w_bcast)
    return combine(y_pad, slot_flat)


# speedup vs baseline: 2.0937x; 1.0400x over previous
import functools

import jax
import jax.numpy as jnp
from jax import lax
from jax.experimental import pallas as pl
from jax.experimental.pallas import tpu as pltpu
from jax.experimental.pallas import tpu_sc as plsc

E = 8
K = 2
T = 4096
H = 1024
I = 4096
BM = 256
NB = T * K // BM + E
NG = 4
NB2 = NB // NG
S_PAD = NB * BM
C = 1024
NJ = I // C
NBPAD = 64
CH = 512
D_SUB = 32
C_SUB = 16


def _routing_body(logits_ref, slot_ref, wrep_ref, bexp_ref):
    logits = logits_ref[...]
    m = jnp.max(logits, axis=1, keepdims=True)
    ex = jnp.exp(logits - m)
    p = ex / jnp.sum(ex, axis=1, keepdims=True)
    ei = lax.broadcasted_iota(jnp.int32, (T, E), 1)
    p1 = jnp.max(p, axis=1, keepdims=True)
    id1 = jnp.min(jnp.where(p == p1, ei, E), axis=1, keepdims=True)
    oh1 = (ei == id1).astype(jnp.float32)
    pm = jnp.where(ei == id1, -1.0, p)
    p2 = jnp.max(pm, axis=1, keepdims=True)
    id2 = jnp.min(jnp.where(pm == p2, ei, E), axis=1, keepdims=True)
    oh2 = (ei == id2).astype(jnp.float32)
    wsum = p1 + p2
    w1 = p1 / wsum
    w2 = p2 / wsum

    oh_tok = oh1 + oh2
    stril = (lax.broadcasted_iota(jnp.int32, (CH, CH), 1)
             < lax.broadcasted_iota(jnp.int32, (CH, CH), 0)).astype(jnp.float32)
    base = jnp.zeros((1, E), jnp.float32)
    prefs = []
    for ci in range(T // CH):
        blk = oh_tok[ci * CH:(ci + 1) * CH, :]
        within = lax.dot_general(stril, blk, (((1,), (0,)), ((), ())),
                                 preferred_element_type=jnp.float32)
        prefs.append(within + base)
        base = base + jnp.sum(blk, axis=0, keepdims=True)
    prefix = jnp.concatenate(prefs, axis=0)

    counts = base
    blocks = jnp.floor((counts + (BM - 1)) / BM)
    tri = (lax.broadcasted_iota(jnp.int32, (E, E), 0)
           <= lax.broadcasted_iota(jnp.int32, (E, E), 1)).astype(jnp.float32)
    cum = lax.dot_general(blocks, tri, (((1,), (0,)), ((), ())),
                          preferred_element_type=jnp.float32)
    offset = (cum - blocks) * BM

    r1 = jnp.sum(prefix * oh1, axis=1)
    r2 = jnp.sum(prefix * oh2, axis=1)
    o1 = jnp.sum(offset * oh1, axis=1)
    o2 = jnp.sum(offset * oh2, axis=1)
    slot1 = (o1 + r1).astype(jnp.int32)
    slot2 = (o2 + r2).astype(jnp.int32)
    slot_ref[...] = jnp.concatenate(
        [slot1.reshape(1, T), slot2.reshape(1, T)], axis=0)
    wrep_ref[...] = jnp.concatenate(
        [jnp.broadcast_to(w1, (T, 128)), jnp.broadcast_to(w2, (T, 128))],
        axis=0)

    bi = lax.broadcasted_iota(jnp.int32, (NBPAD, E), 0).astype(jnp.float32)
    be = jnp.sum((bi >= jnp.broadcast_to(cum, (NBPAD, E))).astype(jnp.int32),
                 axis=1)
    berow = jnp.minimum(be, E - 1).reshape(1, NBPAD)
    nbrow = jnp.broadcast_to(cum[0:1, E - 1:E], (1, NBPAD)).astype(jnp.int32)
    bexp_ref[...] = jnp.concatenate([berow, nbrow] * 4, axis=0)


def _routing(router_logits):
    return pl.pallas_call(
        _routing_body,
        out_shape=(
            jax.ShapeDtypeStruct((K, T), jnp.int32),
            jax.ShapeDtypeStruct((K * T, 128), jnp.float32),
            jax.ShapeDtypeStruct((8, NBPAD), jnp.int32),
        ),
    )(router_logits)


def _dispatch_body(hidden_hbm, slot_hbm, wrep_hbm, xpad_hbm, wb_hbm,
                   idx_v, rows_v, wrows_v, s0, s1, s2, s3):
    sem = [s0, s1]
    sem2 = [s2, s3]
    c = lax.axis_index("c")
    s = lax.axis_index("s")
    wid = s * 2 + c
    k = wid % 2
    part = wid // 2
    per_part = T // 16
    n_u = per_part // D_SUB
    pend = [None, None]
    for u in range(n_u):
        b = u % 2
        if pend[b] is not None:
            pend[b][0].wait()
            pend[b][1].wait()
        tbase = part * per_part + u * D_SUB
        pltpu.sync_copy(slot_hbm.at[pl.ds(k * T + tbase, D_SUB)], idx_v.at[b])
        pltpu.sync_copy(hidden_hbm.at[pl.ds(tbase, D_SUB)], rows_v.at[b])
        pltpu.sync_copy(wrep_hbm.at[pl.ds(k * T + tbase, D_SUB)], wrows_v.at[b])
        cp1 = pltpu.async_copy(rows_v.at[b], xpad_hbm.at[idx_v.at[b]], sem[b])
        cp2 = pltpu.async_copy(wrows_v.at[b], wb_hbm.at[idx_v.at[b]], sem2[b])
        pend[b] = (cp1, cp2)
    for b in range(2):
        if pend[b] is not None:
            pend[b][0].wait()
            pend[b][1].wait()


def _gmm_body(bexp_ref, x_ref, wg_ref, wu_ref, w2_ref, wb_ref, y_ref):
    g = pl.program_id(0)
    j = pl.program_id(1)
    b = pl.program_id(2)

    @pl.when(g * NB2 + b < bexp_ref[1, 0])
    def _active():
        x = x_ref[...]
        gate = lax.dot_general(x, wg_ref[0], (((1,), (1,)), ((), ())),
                               preferred_element_type=jnp.float32)
        up = lax.dot_general(x, wu_ref[0], (((1,), (1,)), ((), ())),
                             preferred_element_type=jnp.float32)
        act = gate * jax.nn.sigmoid(gate) * up
        y = lax.dot_general(act, w2_ref[0], (((1,), (1,)), ((), ())),
                            preferred_element_type=jnp.float32)
        y = y * wb_ref[:, 0:1]
        rows = pl.ds(b * BM, BM)

        @pl.when(j == 0)
        def _():
            y_ref[rows, :] = y

        @pl.when(j != 0)
        def _():
            y_ref[rows, :] = y_ref[rows, :] + y


def _bcl(g, b, be):
    return jnp.minimum(g * NB2 + b, be[1, 0] - 1)


def _gmm(bexp, x_pad, w13, w2, w_bcast):
    grid_spec = pltpu.PrefetchScalarGridSpec(
        num_scalar_prefetch=1,
        grid=(NG, NJ, NB2),
        in_specs=[
            pl.BlockSpec((BM, H),
                         lambda g, j, b, be: (_bcl(g, b, be), 0)),
            pl.BlockSpec((1, C, H),
                         lambda g, j, b, be: (be[0, _bcl(g, b, be)], j, 0)),
            pl.BlockSpec((1, C, H),
                         lambda g, j, b, be: (be[0, _bcl(g, b, be)], NJ + j, 0)),
            pl.BlockSpec((1, H, C),
                         lambda g, j, b, be: (be[0, _bcl(g, b, be)], 0, j)),
            pl.BlockSpec((BM, 128),
                         lambda g, j, b, be: (_bcl(g, b, be), 0)),
        ],
        out_specs=pl.BlockSpec((S_PAD // NG, H), lambda g, j, b, be: (g, 0)),
    )
    return pl.pallas_call(
        _gmm_body,
        grid_spec=grid_spec,
        out_shape=jax.ShapeDtypeStruct((S_PAD, H), jnp.float32),
        compiler_params=pltpu.CompilerParams(
            dimension_semantics=("arbitrary", "arbitrary", "arbitrary")),
    )(bexp, x_pad, w13, w13, w2, w_bcast)


def _combine_body(y_hbm, slot_hbm, out_hbm, i0_v, i1_v, a_v, b_v,
                  sg0, sg1, sg2, sg3, sw0, sw1):
    semg = [(sg0, sg1), (sg2, sg3)]
    semw = [sw0, sw1]
    c = lax.axis_index("c")
    s = lax.axis_index("s")
    wid = s * 2 + c
    per_tile = T // 32
    n_u = per_tile // C_SUB

    def issue(u, b):
        tbase = wid * per_tile + u * C_SUB
        pltpu.sync_copy(slot_hbm.at[pl.ds(tbase, C_SUB)], i0_v.at[b])
        pltpu.sync_copy(slot_hbm.at[pl.ds(T + tbase, C_SUB)], i1_v.at[b])
        cp0 = pltpu.async_copy(y_hbm.at[i0_v.at[b]], a_v.at[b], semg[b][0])
        cp1 = pltpu.async_copy(y_hbm.at[i1_v.at[b]], b_v.at[b], semg[b][1])
        return cp0, cp1

    pend_g = [None, None]
    pend_w = [None, None]
    pend_g[0] = issue(0, 0)
    for u in range(n_u):
        b = u % 2
        pend_g[b][0].wait()
        pend_g[b][1].wait()
        nb = 1 - b
        if u + 1 < n_u:
            if pend_w[nb] is not None:
                pend_w[nb].wait()
                pend_w[nb] = None
            pend_g[nb] = issue(u + 1, nb)

        def row_body(r, _):
            def col_body(ci, _2):
                for q in range(8):
                    off = pl.multiple_of(ci * 128 + q * 16, 16)
                    plsc.addupdate(a_v.at[b, r, pl.ds(off, 16)],
                                   b_v[b, r, pl.ds(off, 16)])
                return 0
            return lax.fori_loop(0, H // 128, col_body, 0)

        lax.fori_loop(0, C_SUB, row_body, 0)
        tbase = wid * per_tile + u * C_SUB
        pend_w[b] = pltpu.async_copy(a_v.at[b], out_hbm.at[pl.ds(tbase, C_SUB)],
                                     semw[b])
    for b in range(2):
        if pend_w[b] is not None:
            pend_w[b].wait()


@functools.cache
def _sc_kernels():
    mesh = plsc.VectorSubcoreMesh(core_axis_name="c", subcore_axis_name="s")
    dispatch = pl.kernel(
        _dispatch_body,
        out_type=(jax.ShapeDtypeStruct((S_PAD, H), jnp.float32),
                  jax.ShapeDtypeStruct((S_PAD, 128), jnp.float32)),
        mesh=mesh,
        scratch_types=[pltpu.VMEM((2, D_SUB), jnp.int32),
                       pltpu.VMEM((2, D_SUB, H), jnp.float32),
                       pltpu.VMEM((2, D_SUB, 128), jnp.float32),
                       pltpu.SemaphoreType.DMA,
                       pltpu.SemaphoreType.DMA,
                       pltpu.SemaphoreType.DMA,
                       pltpu.SemaphoreType.DMA],
    )
    combine = pl.kernel(
        _combine_body,
        out_type=jax.ShapeDtypeStruct((T, H), jnp.float32),
        mesh=mesh,
        scratch_types=[pltpu.VMEM((2, C_SUB), jnp.int32),
                       pltpu.VMEM((2, C_SUB), jnp.int32),
                       pltpu.VMEM((2, C_SUB, H), jnp.float32),
                       pltpu.VMEM((2, C_SUB, H), jnp.float32),
                       pltpu.SemaphoreType.DMA,
                       pltpu.SemaphoreType.DMA,
                       pltpu.SemaphoreType.DMA,
                       pltpu.SemaphoreType.DMA,
                       pltpu.SemaphoreType.DMA,
                       pltpu.SemaphoreType.DMA],
    )
    return dispatch, combine


def kernel(hidden_states, router_logits, w13_weight, w2_weight):
    dispatch, combine = _sc_kernels()
    slot, wrep, bexp8 = _routing(router_logits)
    slot_flat = slot.reshape(K * T)
    x_pad, w_bcast = dispatch(hidden_states, slot_flat, wrep)
    y_pad = _gmm(bexp8, x_pad, w13_weight, w2_weight, w_bcast)
    return combine(y_pad, slot_flat)
